# baseline (speedup 1.0000x reference)
import functools

import jax
import jax.numpy as jnp
from jax import lax
from jax.experimental import pallas as pl
from jax.experimental.pallas import tpu as pltpu
from jax.experimental.pallas import tpu_sc as plsc

EMB_D = 64
IDX_CHUNK = 128


def _make_sc_kernel(num_workers, b_per_w, n_chunks):
    mesh = plsc.VectorSubcoreMesh(core_axis_name="c", subcore_axis_name="s")
    num_cores = 2

    @functools.partial(
        pl.kernel,
        out_type=jax.ShapeDtypeStruct((num_workers, b_per_w, 2, EMB_D),
                                      jnp.float32),
        mesh=mesh,
        scratch_types=[
            pltpu.VMEM((n_chunks, IDX_CHUNK), jnp.int32),
            pltpu.VMEM((b_per_w, EMB_D), jnp.float32),
            pltpu.VMEM((b_per_w, EMB_D), jnp.float32),
            pltpu.SemaphoreType.DMA,
        ],
        compiler_params=pltpu.CompilerParams(use_tc_tiling_on_sc=False),
    )
    def emb_combine(x_hbm, poi_hbm, loc_hbm, out_hbm, idx_v, poi_v, loc_v,
                    sem):
        wid = lax.axis_index("s") * num_cores + lax.axis_index("c")
        pltpu.sync_copy(x_hbm.at[wid], idx_v)
        copies = []
        for j in range(n_chunks):
            rows = pl.ds(j * IDX_CHUNK, IDX_CHUNK)
            copies.append(
                pltpu.async_copy(poi_hbm.at[idx_v.at[j]], poi_v.at[rows], sem))
            copies.append(
                pltpu.async_copy(loc_hbm.at[idx_v.at[j]], loc_v.at[rows], sem))
        for c in copies:
            c.wait()
        pltpu.sync_copy(poi_v, out_hbm.at[wid, :, 0, :])
        pltpu.sync_copy(loc_v, out_hbm.at[wid, :, 1, :])

    return emb_combine


def kernel(x, poi_table, loc_table):
    b = x.shape[0]
    info = plsc.get_sparse_core_info()
    num_workers = info.num_cores * info.num_subcores
    b_per_w = b // num_workers
    n_chunks = b_per_w // IDX_CHUNK
    x2 = x.reshape(num_workers, n_chunks, IDX_CHUNK).astype(jnp.int32)
    out = _make_sc_kernel(num_workers, b_per_w, n_chunks)(
        x2, poi_table, loc_table)
    return out.reshape(b, 2 * EMB_D)
